# Initial kernel scaffold; baseline (speedup 1.0000x reference)
import functools

import jax
import jax.numpy as jnp
from jax import lax
from jax.experimental import pallas as pl
from jax.experimental.pallas import tpu as pltpu
from jax.experimental.pallas import tpu_sc as plsc

N = 10000
E = 320000
D = 128
H = 256
G = 64
C = 150

NC = 2
NS = 16
NW = NC * NS
CH = 128
CPT = -(-E // (NW * CH))
EPT = CPT * CH
E_PAD = NW * EPT
STRIPE = N // NS

_sc_mesh = plsc.VectorSubcoreMesh(core_axis_name="c", subcore_axis_name="s")


@functools.partial(
    pl.kernel,
    out_type=jax.ShapeDtypeStruct((NC, N, D), jnp.float32),
    mesh=_sc_mesh,
    scratch_types=[
        pltpu.VMEM((CPT, CH), jnp.int32),
        pltpu.VMEM((CPT, CH), jnp.int32),
        pltpu.VMEM((CH, D), jnp.float32),
        pltpu.VMEM_SHARED((N + 8, D), jnp.float32),
        pltpu.SemaphoreType.DMA,
    ],
)
def _sc_scatter_add(x_hbm, src_hbm, dst_hbm, out_hbm, src_v, dst_v, rows_v,
                    agg_sh, sem):
    c = lax.axis_index("c")
    s = lax.axis_index("s")
    wid = c * NS + s

    def _zero_body(i, _):
        r = i // (D // 16)
        col = lax.rem(i, D // 16)
        rows_v[r, pl.ds(col * 16, 16)] = jnp.zeros((16,), jnp.float32)
        return 0

    lax.fori_loop(0, CH * (D // 16), _zero_body, 0)

    base_row = s * STRIPE
    n_full = STRIPE // CH
    rem = STRIPE - n_full * CH
    for k in range(n_full):
        pltpu.sync_copy(rows_v, agg_sh.at[pl.ds(base_row + k * CH, CH)])
    pltpu.sync_copy(rows_v.at[pl.ds(0, rem)],
                    agg_sh.at[pl.ds(base_row + n_full * CH, rem)])
    plsc.subcore_barrier()

    pltpu.sync_copy(src_hbm.at[wid], src_v)
    pltpu.sync_copy(dst_hbm.at[wid], dst_v)

    def _edge_body(j, _):
        pltpu.async_copy(x_hbm.at[src_v.at[j]], rows_v, sem).wait()
        pltpu.sync_copy(rows_v, agg_sh.at[dst_v.at[j]], add=True)
        return 0

    lax.fori_loop(0, CPT, _edge_body, 0)
    plsc.subcore_barrier()

    pltpu.sync_copy(agg_sh.at[pl.ds(base_row, STRIPE)],
                    out_hbm.at[c].at[pl.ds(base_row, STRIPE)])


BN = 1000
NB = N // BN


def _tc_k1_body(x_b, a0_b, a1_b, wg, bg, wc, bc, batch_b, h_out, gmax_out):
    i = pl.program_id(0)
    xs = x_b[...] + a0_b[...] + a1_b[...]
    h = jnp.dot(xs, wg[...], preferred_element_type=jnp.float32) + bg[...]
    h_out[...] = h
    xg = jnp.maximum(jnp.dot(h, wc[...], preferred_element_type=jnp.float32)
                     + bc[...], 0.0)

    @pl.when(i == 0)
    def _():
        gmax_out[...] = jnp.zeros_like(gmax_out)

    bcol = batch_b[...]
    gmin = jnp.min(bcol).astype(jnp.int32)
    gmax_id = jnp.max(bcol).astype(jnp.int32)

    def _g_body(g, _):
        mask = bcol == g.astype(jnp.float32)
        m = jnp.max(jnp.where(mask, xg, 0.0), axis=0, keepdims=True)
        gmax_out[pl.ds(g, 1), :] = jnp.maximum(gmax_out[pl.ds(g, 1), :], m)
        return 0

    lax.fori_loop(gmin, gmax_id + 1, _g_body, 0)


def _tc_k2_body(h_b, gmax_ref, batch_b, wt, bt, wo, bo, out_b):
    z = jnp.dot(gmax_ref[...], wt[pl.ds(H, H), :],
                preferred_element_type=jnp.float32)
    bcol = batch_b[...]
    onehot = (bcol == lax.broadcasted_iota(jnp.float32, (1, G), 1)
              ).astype(jnp.float32)
    zb = jnp.dot(onehot, z, preferred_element_type=jnp.float32)
    t = jnp.maximum(
        jnp.dot(h_b[...], wt[pl.ds(0, H), :],
                preferred_element_type=jnp.float32) + zb + bt[...], 0.0)
    out_b[...] = jnp.dot(t, wo[...], preferred_element_type=jnp.float32) + bo[...]


def kernel(x, edge_indices, batch, W_gcn, b_gcn, W_conv, b_conv, W_t, b_t,
           W_out, b_out):
    src = edge_indices[0]
    dst = edge_indices[1]
    pad = E_PAD - E
    src_p = jnp.concatenate([src, jnp.zeros((pad,), jnp.int32)])
    dst_p = jnp.concatenate([dst, jnp.full((pad,), N, jnp.int32)])
    src3 = src_p.reshape(NW, CPT, CH)
    dst3 = dst_p.reshape(NW, CPT, CH)

    agg2 = _sc_scatter_add(x, src3, dst3)

    batch_col = batch.astype(jnp.float32).reshape(N, 1)

    h, gmax = pl.pallas_call(
        _tc_k1_body,
        grid=(NB,),
        in_specs=[
            pl.BlockSpec((BN, D), lambda i: (i, 0)),
            pl.BlockSpec((BN, D), lambda i: (i, 0)),
            pl.BlockSpec((BN, D), lambda i: (i, 0)),
            pl.BlockSpec((D, H), lambda i: (0, 0)),
            pl.BlockSpec((1, H), lambda i: (0, 0)),
            pl.BlockSpec((H, H), lambda i: (0, 0)),
            pl.BlockSpec((1, H), lambda i: (0, 0)),
            pl.BlockSpec((BN, 1), lambda i: (i, 0)),
        ],
        out_specs=[
            pl.BlockSpec((BN, H), lambda i: (i, 0)),
            pl.BlockSpec((G, H), lambda i: (0, 0)),
        ],
        out_shape=[
            jax.ShapeDtypeStruct((N, H), jnp.float32),
            jax.ShapeDtypeStruct((G, H), jnp.float32),
        ],
    )(x, agg2[0], agg2[1], W_gcn, b_gcn.reshape(1, H), W_conv,
      b_conv.reshape(1, H), batch_col)

    out = pl.pallas_call(
        _tc_k2_body,
        grid=(NB,),
        in_specs=[
            pl.BlockSpec((BN, H), lambda i: (i, 0)),
            pl.BlockSpec((G, H), lambda i: (0, 0)),
            pl.BlockSpec((BN, 1), lambda i: (i, 0)),
            pl.BlockSpec((2 * H, 1024), lambda i: (0, 0)),
            pl.BlockSpec((1, 1024), lambda i: (0, 0)),
            pl.BlockSpec((1024, C), lambda i: (0, 0)),
            pl.BlockSpec((1, C), lambda i: (0, 0)),
        ],
        out_specs=pl.BlockSpec((BN, C), lambda i: (i, 0)),
        out_shape=jax.ShapeDtypeStruct((N, C), jnp.float32),
    )(h, gmax, batch_col, W_t, b_t.reshape(1, 1024), W_out,
      b_out.reshape(1, C))

    return out


# speedup vs baseline: 4.5696x; 4.5696x over previous
import functools

import jax
import jax.numpy as jnp
from jax import lax
from jax.experimental import pallas as pl
from jax.experimental.pallas import tpu as pltpu
from jax.experimental.pallas import tpu_sc as plsc

N = 10000
E = 320000
D = 128
H = 256
G = 64
C = 150

NC = 2
NS = 16
NW = NC * NS
CH = 128
CPT = -(-E // (NW * CH))
EPT = CPT * CH
E_PAD = NW * EPT
SPT = 632
NP = SPT * NS

def _sc_scatter_add_body(x_hbm, src_hbm, dst_hbm, out_hbm, src_v, dst_v,
                         rows_v, agg_sh, sem):
    c = lax.axis_index("c")
    s = lax.axis_index("s")
    wid = c * NS + s

    def _zero_body(i, _):
        r = i // (D // 16)
        col = lax.rem(i, D // 16)
        rows_v[r, pl.ds(col * 16, 16)] = jnp.zeros((16,), jnp.float32)
        return 0

    lax.fori_loop(0, CH * (D // 16), _zero_body, 0)

    base_row = s * SPT
    n_full = SPT // CH
    rem = SPT - n_full * CH
    for k in range(n_full):
        pltpu.sync_copy(rows_v, agg_sh.at[pl.ds(base_row + k * CH, CH)])
    pltpu.sync_copy(rows_v.at[pl.ds(0, rem)],
                    agg_sh.at[pl.ds(base_row + n_full * CH, rem)])
    plsc.subcore_barrier()

    pltpu.sync_copy(src_hbm.at[wid], src_v)
    pltpu.sync_copy(dst_hbm.at[wid], dst_v)

    def _edge_body(j, _):
        pltpu.async_copy(x_hbm.at[src_v.at[j]], rows_v, sem).wait()
        pltpu.sync_copy(rows_v, agg_sh.at[dst_v.at[j]], add=True)
        return 0

    lax.fori_loop(0, CPT, _edge_body, 0)
    plsc.subcore_barrier()

    pltpu.sync_copy(agg_sh.at[pl.ds(base_row, SPT)],
                    out_hbm.at[c].at[pl.ds(base_row, SPT)])


@functools.cache
def _sc_scatter_add():
    mesh = plsc.VectorSubcoreMesh(core_axis_name="c", subcore_axis_name="s",
                                  num_cores=NC, num_subcores=NS)
    return pl.kernel(
        _sc_scatter_add_body,
        out_type=jax.ShapeDtypeStruct((NC, NP, D), jnp.float32),
        mesh=mesh,
        scratch_types=[
            pltpu.VMEM((CPT, CH), jnp.int32),
            pltpu.VMEM((CPT, CH), jnp.int32),
            pltpu.VMEM((CH, D), jnp.float32),
            pltpu.VMEM_SHARED((NP, D), jnp.float32),
            pltpu.SemaphoreType.DMA,
        ],
    )


BN = 1000
NB = N // BN


def _tc_k1_body(x_b, a0_b, a1_b, wg, bg, wc, bc, batch_b, h_out, gmax_out):
    i = pl.program_id(0)
    xs = x_b[...] + a0_b[...] + a1_b[...]
    h = jnp.dot(xs, wg[...], preferred_element_type=jnp.float32) + bg[...]
    h_out[...] = h
    xg = jnp.maximum(jnp.dot(h, wc[...], preferred_element_type=jnp.float32)
                     + bc[...], 0.0)

    @pl.when(i == 0)
    def _():
        gmax_out[...] = jnp.zeros_like(gmax_out)

    bcol = batch_b[...]
    gmin = jnp.min(bcol).astype(jnp.int32)
    gmax_id = jnp.max(bcol).astype(jnp.int32)

    def _g_body(g, _):
        mask = bcol == g.astype(jnp.float32)
        m = jnp.max(jnp.where(mask, xg, 0.0), axis=0, keepdims=True)
        gmax_out[pl.ds(g, 1), :] = jnp.maximum(gmax_out[pl.ds(g, 1), :], m)
        return 0

    lax.fori_loop(gmin, gmax_id + 1, _g_body, 0)


def _tc_k2_body(h_b, gmax_ref, batch_b, wt, bt, wo, bo, out_b):
    z = jnp.dot(gmax_ref[...], wt[pl.ds(H, H), :],
                preferred_element_type=jnp.float32)
    bcol = batch_b[...]
    onehot = (bcol == lax.broadcasted_iota(jnp.int32, (1, G), 1)
              .astype(jnp.float32)).astype(jnp.float32)
    zb = jnp.dot(onehot, z, preferred_element_type=jnp.float32)
    t = jnp.maximum(
        jnp.dot(h_b[...], wt[pl.ds(0, H), :],
                preferred_element_type=jnp.float32) + zb + bt[...], 0.0)
    out_b[...] = jnp.dot(t, wo[...], preferred_element_type=jnp.float32) + bo[...]


def kernel(x, edge_indices, batch, W_gcn, b_gcn, W_conv, b_conv, W_t, b_t,
           W_out, b_out):
    src = edge_indices[0]
    dst = edge_indices[1]
    pad = E_PAD - E
    src_p = jnp.concatenate([src, jnp.zeros((pad,), jnp.int32)])
    dst_p = jnp.concatenate([dst, jnp.full((pad,), N, jnp.int32)])
    src3 = src_p.reshape(NW, CPT, CH)
    dst3 = dst_p.reshape(NW, CPT, CH)

    agg2 = _sc_scatter_add()(x, src3, dst3)

    batch_col = batch.astype(jnp.float32).reshape(N, 1)

    h, gmax = pl.pallas_call(
        _tc_k1_body,
        grid=(NB,),
        in_specs=[
            pl.BlockSpec((BN, D), lambda i: (i, 0)),
            pl.BlockSpec((BN, D), lambda i: (i, 0)),
            pl.BlockSpec((BN, D), lambda i: (i, 0)),
            pl.BlockSpec((D, H), lambda i: (0, 0)),
            pl.BlockSpec((1, H), lambda i: (0, 0)),
            pl.BlockSpec((H, H), lambda i: (0, 0)),
            pl.BlockSpec((1, H), lambda i: (0, 0)),
            pl.BlockSpec((BN, 1), lambda i: (i, 0)),
        ],
        out_specs=[
            pl.BlockSpec((BN, H), lambda i: (i, 0)),
            pl.BlockSpec((G, H), lambda i: (0, 0)),
        ],
        out_shape=[
            jax.ShapeDtypeStruct((N, H), jnp.float32),
            jax.ShapeDtypeStruct((G, H), jnp.float32),
        ],
    )(x, agg2[0, :N], agg2[1, :N], W_gcn, b_gcn.reshape(1, H), W_conv,
      b_conv.reshape(1, H), batch_col)

    out = pl.pallas_call(
        _tc_k2_body,
        grid=(NB,),
        in_specs=[
            pl.BlockSpec((BN, H), lambda i: (i, 0)),
            pl.BlockSpec((G, H), lambda i: (0, 0)),
            pl.BlockSpec((BN, 1), lambda i: (i, 0)),
            pl.BlockSpec((2 * H, 1024), lambda i: (0, 0)),
            pl.BlockSpec((1, 1024), lambda i: (0, 0)),
            pl.BlockSpec((1024, C), lambda i: (0, 0)),
            pl.BlockSpec((1, C), lambda i: (0, 0)),
        ],
        out_specs=pl.BlockSpec((BN, C), lambda i: (i, 0)),
        out_shape=jax.ShapeDtypeStruct((N, C), jnp.float32),
    )(h, gmax, batch_col, W_t, b_t.reshape(1, 1024), W_out,
      b_out.reshape(1, C))

    return out
